# baseline (speedup 1.0000x reference)
import dataclasses
import functools
import jax
import jax.numpy as jnp
import numpy as np
from jax import lax
from jax.experimental import pallas as pl
from jax.experimental.pallas import tpu as pltpu
from jax.experimental.pallas import tpu_sc as plsc

N_NODES = 10000
N_EDGES = 320000
D = 128
DH2 = 64
H = 4
D_H = 32

NP = 10240
NS = 16
CHUNK = 128
CHUNKS_PER_S = 158
EP = NS * CHUNKS_PER_S * CHUNK
NP_SC = 10016
ROWS_PER_TILE = NP_SC // NS
ZROWS = 313
NBLK = 256
LEAK = 0.2


def _tc1_body(x_ref, w_ref, asrc_ref, adst_ref, wx_ref, ssrc_ref, sdst_ref):
    b = pl.program_id(0)
    wx = jnp.dot(x_ref[...], w_ref[...], preferred_element_type=jnp.float32)
    ssrc = jnp.dot(wx, asrc_ref[...], preferred_element_type=jnp.float32)
    sdst = jnp.dot(wx, adst_ref[...], preferred_element_type=jnp.float32)
    rows = b * NBLK + lax.broadcasted_iota(jnp.int32, (NBLK, 1), 0)
    ssrc = jnp.where(rows < N_NODES, ssrc, jnp.float32(-1e30))
    wx_ref[...] = wx
    ssrc_ref[...] = ssrc
    sdst_ref[...] = sdst


def _tc2_body(a0_ref, a1_ref, z0_ref, z1_ref, m_ref, wot_ref, bo_ref, out_ref):
    agg = jnp.concatenate([a0_ref[...], a1_ref[...]], axis=1)
    z = z0_ref[...] + z1_ref[...]
    d = jnp.dot(z, m_ref[...], preferred_element_type=jnp.float32) + 1e-16
    o = jnp.dot(agg / d, wot_ref[...], preferred_element_type=jnp.float32)
    o = o + bo_ref[...]
    out_ref[...] = jnp.where(o > 0, o, jnp.exp(o) - 1.0)


def _sc_body(tsrc_h, tdst_h, wxs_h, src_h, dst_h, za_h,
             aggz_out,
             si_v, di_v, tsrc_v, tdst_v, gwx, gz, aggz_s):
    c = lax.axis_index("c")
    s = lax.axis_index("s")

    @pl.loop(0, ROWS_PER_TILE // ZROWS)
    def _zero(r):
        base = s * ROWS_PER_TILE + r * ZROWS
        pltpu.sync_copy(za_h, aggz_s.at[pl.ds(base, ZROWS)])

    pltpu.sync_copy(tsrc_h.at[c], tsrc_v)
    pltpu.sync_copy(tdst_h.at[c], tdst_v)

    zeros16 = jnp.zeros((16,), jnp.float32)

    @pl.loop(0, CHUNK)
    def _wz(i):
        gz[i, pl.ds(DH2, 16)] = zeros16

    plsc.subcore_barrier()

    lane = lax.iota(jnp.int32, 16)
    h0 = c * 2
    lane_h = [jnp.full((16,), DH2 + h0 + hh, jnp.int32) for hh in range(2)]

    def _splat(v, idx):
        return lax.gather(
            v, idx[:, None],
            lax.GatherDimensionNumbers(
                offset_dims=(), collapsed_slice_dims=(0,),
                start_index_map=(0,)),
            (1,), mode=lax.GatherScatterMode.PROMISE_IN_BOUNDS)

    @pl.loop(0, CHUNKS_PER_S)
    def _chunk(cc):
        pltpu.sync_copy(src_h.at[s].at[cc], si_v)
        pltpu.sync_copy(dst_h.at[s].at[cc], di_v)
        pltpu.sync_copy(wxs_h.at[c].at[si_v], gwx)

        for j in range(CHUNK // 16):
            s16 = si_v[pl.ds(j * 16, 16)]
            d16 = di_v[pl.ds(j * 16, 16)]
            s2 = s16 + s16
            d2 = d16 + d16
            wregs = []
            for hh in range(2):
                siv = plsc.load_gather(tsrc_v, [s2 + hh])
                sjv = plsc.load_gather(tdst_v, [d2 + hh])
                es = siv + sjv
                e = jnp.maximum(es, es * LEAK)
                w16 = jnp.exp(e)
                wregs.append(w16)
                plsc.store_scatter(gz, [lane + j * 16, lane_h[hh]], w16)
            w0, w1 = wregs

            @pl.loop(0, 16, step=4)
            def _edge(kk):
                for u in range(4):
                    k = kk + u
                    ck = jnp.full((16,), k, jnp.int32)
                    bc0 = _splat(w0, ck)
                    bc1 = _splat(w1, ck)
                    i = j * 16 + k
                    for hh, bc in ((0, bc0), (1, bc1)):
                        for q in range(2):
                            sl = pl.ds(hh * 32 + q * 16, 16)
                            gz[i, sl] = gwx[i, sl] * bc

        pltpu.sync_copy(gz, aggz_s.at[di_v], add=True)

    plsc.subcore_barrier()
    stripe = pl.ds(s * ROWS_PER_TILE, ROWS_PER_TILE)
    pltpu.sync_copy(aggz_s.at[stripe], aggz_out.at[c, stripe])


def kernel(x, edge_index, W, attn_vec, Wo, bo):
    f32 = jnp.float32

    asrc = jnp.zeros((D, 16), f32)
    adst = jnp.zeros((D, 16), f32)
    for h in range(H):
        asrc = asrc.at[h * D_H:(h + 1) * D_H, h].set(attn_vec[h, :D_H])
        adst = adst.at[h * D_H:(h + 1) * D_H, h].set(attn_vec[h, D_H:])
    m16 = np.zeros((16, D), np.float32)
    for l in range(D):
        m16[l // D_H, l] = 1.0
    m16 = jnp.asarray(m16)

    xp = jnp.concatenate([x, jnp.zeros((NP - N_NODES, D), f32)])
    srcp = jnp.concatenate(
        [edge_index[0], jnp.full((EP - N_EDGES,), N_NODES, jnp.int32)]
    ).reshape(NS, CHUNKS_PER_S, CHUNK)
    dstp = jnp.concatenate(
        [edge_index[1], jnp.full((EP - N_EDGES,), N_NODES, jnp.int32)]
    ).reshape(NS, CHUNKS_PER_S, CHUNK)

    grid1 = (NP // NBLK,)
    wxp, ssrcp, sdstp = pl.pallas_call(
        _tc1_body,
        grid=grid1,
        in_specs=[
            pl.BlockSpec((NBLK, D), lambda b: (b, 0)),
            pl.BlockSpec((D, D), lambda b: (0, 0)),
            pl.BlockSpec((D, 16), lambda b: (0, 0)),
            pl.BlockSpec((D, 16), lambda b: (0, 0)),
        ],
        out_specs=[
            pl.BlockSpec((NBLK, D), lambda b: (b, 0)),
            pl.BlockSpec((NBLK, 16), lambda b: (b, 0)),
            pl.BlockSpec((NBLK, 16), lambda b: (b, 0)),
        ],
        out_shape=[
            jax.ShapeDtypeStruct((NP, D), f32),
            jax.ShapeDtypeStruct((NP, 16), f32),
            jax.ShapeDtypeStruct((NP, 16), f32),
        ],
    )(xp, W, asrc, adst)

    wxsplit = jnp.stack([wxp[:, :DH2], wxp[:, DH2:]])
    tsrc = jnp.stack([ssrcp[:NP_SC, 0:2].reshape(-1),
                      ssrcp[:NP_SC, 2:4].reshape(-1)])
    tdst = jnp.stack([sdstp[:NP_SC, 0:2].reshape(-1),
                      sdstp[:NP_SC, 2:4].reshape(-1)])

    mesh = plsc.VectorSubcoreMesh(core_axis_name="c", subcore_axis_name="s")
    cp = pltpu.CompilerParams(
        needs_layout_passes=False, use_tc_tiling_on_sc=False
    )
    sc_kernel = pl.kernel(
        _sc_body,
        compiler_params=cp,
        out_type=jax.ShapeDtypeStruct((2, NP_SC, DH2 + 16), f32),
        mesh=mesh,
        scratch_types=[
            pltpu.VMEM((CHUNK,), jnp.int32),
            pltpu.VMEM((CHUNK,), jnp.int32),
            pltpu.VMEM((NP_SC * 2,), f32),
            pltpu.VMEM((NP_SC * 2,), f32),
            pltpu.VMEM((CHUNK, DH2), f32),
            pltpu.VMEM((CHUNK, DH2 + 16), f32),
            pltpu.VMEM_SHARED((NP_SC, DH2 + 16), f32),
        ],
    )
    za = jnp.zeros((ZROWS, DH2 + 16), f32)
    aggz = sc_kernel(tsrc, tdst, wxsplit, srcp, dstp, za)
    aggz = jnp.concatenate(
        [aggz, jnp.zeros((2, NP - NP_SC, DH2 + 16), f32)], axis=1)
    agg = aggz[:, :, :DH2]
    z = aggz[:, :, DH2:]

    grid3 = (NP // NBLK,)
    out = pl.pallas_call(
        _tc2_body,
        grid=grid3,
        in_specs=[
            pl.BlockSpec((NBLK, DH2), lambda b: (b, 0)),
            pl.BlockSpec((NBLK, DH2), lambda b: (b, 0)),
            pl.BlockSpec((NBLK, 16), lambda b: (b, 0)),
            pl.BlockSpec((NBLK, 16), lambda b: (b, 0)),
            pl.BlockSpec((16, D), lambda b: (0, 0)),
            pl.BlockSpec((D, D), lambda b: (0, 0)),
            pl.BlockSpec((1, D), lambda b: (0, 0)),
        ],
        out_specs=pl.BlockSpec((NBLK, D), lambda b: (b, 0)),
        out_shape=jax.ShapeDtypeStruct((NP, D), f32),
    )(agg[0], agg[1], z[0], z[1], m16, Wo.T, bo.reshape(1, D))

    return out[:N_NODES]


# speedup vs baseline: 1.2370x; 1.2370x over previous
import dataclasses
import functools
import jax
import jax.numpy as jnp
import numpy as np
from jax import lax
from jax.experimental import pallas as pl
from jax.experimental.pallas import tpu as pltpu
from jax.experimental.pallas import tpu_sc as plsc

N_NODES = 10000
N_EDGES = 320000
D = 128
DH2 = 64
H = 4
D_H = 32

NP = 10240
NS = 16
CHUNK = 128
CHUNKS_PER_S = 158
EP = NS * CHUNKS_PER_S * CHUNK
NP_SC = 10016
ROWS_PER_TILE = NP_SC // NS
ZROWS = 313
NBLK = 256
LEAK = 0.2


def _tc1_body(x_ref, w_ref, asrc_ref, adst_ref, wx_ref, ssrc_ref, sdst_ref):
    b = pl.program_id(0)
    wx = jnp.dot(x_ref[...], w_ref[...], preferred_element_type=jnp.float32)
    ssrc = jnp.dot(wx, asrc_ref[...], preferred_element_type=jnp.float32)
    sdst = jnp.dot(wx, adst_ref[...], preferred_element_type=jnp.float32)
    rows = b * NBLK + lax.broadcasted_iota(jnp.int32, (NBLK, 1), 0)
    ssrc = jnp.where(rows < N_NODES, ssrc, jnp.float32(-1e30))
    wx_ref[...] = wx
    ssrc_ref[...] = ssrc
    sdst_ref[...] = sdst


def _tc2_body(a0_ref, a1_ref, z0_ref, z1_ref, m_ref, wot_ref, bo_ref, out_ref):
    agg = jnp.concatenate([a0_ref[...], a1_ref[...]], axis=1)
    z = z0_ref[...] + z1_ref[...]
    d = jnp.dot(z, m_ref[...], preferred_element_type=jnp.float32) + 1e-16
    o = jnp.dot(agg / d, wot_ref[...], preferred_element_type=jnp.float32)
    o = o + bo_ref[...]
    out_ref[...] = jnp.where(o > 0, o, jnp.exp(o) - 1.0)


def _sc_body(tsrc_h, tdst_h, wxs_h, idx_h, za_h,
             aggz_out,
             sd0, sd1, dscat0, dscat1, gwx0, gwx1, gz0, gz1,
             tsrc_v, tdst_v, aggz_s,
             isem0, isem1, gsem0, gsem1, ssem0, ssem1):
    c = lax.axis_index("c")
    s = lax.axis_index("s")
    sd = (sd0, sd1)
    dscat = (dscat0, dscat1)
    gwx = (gwx0, gwx1)
    gz = (gz0, gz1)
    isem = (isem0, isem1)
    gsem = (gsem0, gsem1)
    ssem = (ssem0, ssem1)

    @pl.loop(0, ROWS_PER_TILE // ZROWS)
    def _zero(r):
        base = s * ROWS_PER_TILE + r * ZROWS
        pltpu.sync_copy(za_h, aggz_s.at[pl.ds(base, ZROWS)])

    pltpu.sync_copy(tsrc_h.at[c], tsrc_v)
    pltpu.sync_copy(tdst_h.at[c], tdst_v)

    zeros16 = jnp.zeros((16,), jnp.float32)

    @pl.loop(0, CHUNK)
    def _wz(i):
        gz0[i, pl.ds(DH2, 16)] = zeros16
        gz1[i, pl.ds(DH2, 16)] = zeros16

    plsc.subcore_barrier()

    lane = lax.iota(jnp.int32, 16)
    h0 = c * 2
    lane_h = [jnp.full((16,), DH2 + h0 + hh, jnp.int32) for hh in range(2)]

    def _splat(v, idx):
        return lax.gather(
            v, idx[:, None],
            lax.GatherDimensionNumbers(
                offset_dims=(), collapsed_slice_dims=(0,),
                start_index_map=(0,)),
            (1,), mode=lax.GatherScatterMode.PROMISE_IN_BOUNDS)

    def _idx_copy(q, b):
        return pltpu.make_async_copy(idx_h.at[s].at[q], sd[b], isem[b])

    def _gather(q, b):
        return pltpu.make_async_copy(
            wxs_h.at[c].at[sd[b].at[0]], gwx[b], gsem[b])

    def _scatter(b):
        return pltpu.make_async_copy(gz[b], aggz_s.at[dscat[b]], ssem[b])

    cp0 = _idx_copy(0, 0)
    cp0.start()
    _idx_copy(1, 1).start()
    cp0.wait()
    _gather(0, 0).start()

    @pl.loop(0, CHUNKS_PER_S, step=2)
    def _chunk(cc):
        for b in range(2):
            q = cc + b
            nb = 1 - b

            @pl.when(q + 1 < CHUNKS_PER_S)
            def _prefetch():
                _idx_copy(q + 1, nb).wait()
                _gather(q + 1, nb).start()

            _gather(q, b).wait()

            @pl.when(q >= 2)
            def _drain():
                _scatter(b).wait()

            gwb = gwx[b]
            gzb = gz[b]
            for j in range(CHUNK // 16):
                s16 = sd[b][0, pl.ds(j * 16, 16)]
                d16 = sd[b][1, pl.ds(j * 16, 16)]
                dscat[b][pl.ds(j * 16, 16)] = d16
                s2 = s16 + s16
                d2 = d16 + d16
                wregs = []
                for hh in range(2):
                    siv = plsc.load_gather(tsrc_v, [s2 + hh])
                    sjv = plsc.load_gather(tdst_v, [d2 + hh])
                    es = siv + sjv
                    e = jnp.maximum(es, es * LEAK)
                    w16 = jnp.exp(e)
                    wregs.append(w16)
                    plsc.store_scatter(
                        gzb, [lane + j * 16, lane_h[hh]], w16)
                w0, w1 = wregs

                @pl.loop(0, 16, step=4)
                def _edge(kk):
                    for u in range(4):
                        k = kk + u
                        ck = jnp.full((16,), k, jnp.int32)
                        bc0 = _splat(w0, ck)
                        bc1 = _splat(w1, ck)
                        i = j * 16 + k
                        for hh, bc in ((0, bc0), (1, bc1)):
                            for qq in range(2):
                                sl = pl.ds(hh * 32 + qq * 16, 16)
                                gzb[i, sl] = gwb[i, sl] * bc

            pltpu.async_copy(gz[b], aggz_s.at[dscat[b]], ssem[b], add=True)

            @pl.when(q + 2 < CHUNKS_PER_S)
            def _nexti():
                _idx_copy(q + 2, b).start()

    _scatter(0).wait()
    _scatter(1).wait()

    plsc.subcore_barrier()
    stripe = pl.ds(s * ROWS_PER_TILE, ROWS_PER_TILE)
    pltpu.sync_copy(aggz_s.at[stripe], aggz_out.at[c, stripe])


def kernel(x, edge_index, W, attn_vec, Wo, bo):
    f32 = jnp.float32

    asrc = jnp.zeros((D, 16), f32)
    adst = jnp.zeros((D, 16), f32)
    for h in range(H):
        asrc = asrc.at[h * D_H:(h + 1) * D_H, h].set(attn_vec[h, :D_H])
        adst = adst.at[h * D_H:(h + 1) * D_H, h].set(attn_vec[h, D_H:])
    m16 = np.zeros((16, D), np.float32)
    for l in range(D):
        m16[l // D_H, l] = 1.0
    m16 = jnp.asarray(m16)

    xp = jnp.concatenate([x, jnp.zeros((NP - N_NODES, D), f32)])
    srcp = jnp.concatenate(
        [edge_index[0], jnp.full((EP - N_EDGES,), N_NODES, jnp.int32)]
    ).reshape(NS, CHUNKS_PER_S, CHUNK)
    dstp = jnp.concatenate(
        [edge_index[1], jnp.full((EP - N_EDGES,), N_NODES, jnp.int32)]
    ).reshape(NS, CHUNKS_PER_S, CHUNK)

    grid1 = (NP // NBLK,)
    wxp, ssrcp, sdstp = pl.pallas_call(
        _tc1_body,
        grid=grid1,
        in_specs=[
            pl.BlockSpec((NBLK, D), lambda b: (b, 0)),
            pl.BlockSpec((D, D), lambda b: (0, 0)),
            pl.BlockSpec((D, 16), lambda b: (0, 0)),
            pl.BlockSpec((D, 16), lambda b: (0, 0)),
        ],
        out_specs=[
            pl.BlockSpec((NBLK, D), lambda b: (b, 0)),
            pl.BlockSpec((NBLK, 16), lambda b: (b, 0)),
            pl.BlockSpec((NBLK, 16), lambda b: (b, 0)),
        ],
        out_shape=[
            jax.ShapeDtypeStruct((NP, D), f32),
            jax.ShapeDtypeStruct((NP, 16), f32),
            jax.ShapeDtypeStruct((NP, 16), f32),
        ],
    )(xp, W, asrc, adst)

    wxsplit = jnp.stack([wxp[:, :DH2], wxp[:, DH2:]])
    tsrc = jnp.stack([ssrcp[:NP_SC, 0:2].reshape(-1),
                      ssrcp[:NP_SC, 2:4].reshape(-1)])
    tdst = jnp.stack([sdstp[:NP_SC, 0:2].reshape(-1),
                      sdstp[:NP_SC, 2:4].reshape(-1)])

    mesh = plsc.VectorSubcoreMesh(core_axis_name="c", subcore_axis_name="s")
    cp = pltpu.CompilerParams(
        needs_layout_passes=False, use_tc_tiling_on_sc=False
    )
    sc_kernel = pl.kernel(
        _sc_body,
        compiler_params=cp,
        out_type=jax.ShapeDtypeStruct((2, NP_SC, DH2 + 16), f32),
        mesh=mesh,
        scratch_types=[
            pltpu.VMEM((2, CHUNK), jnp.int32),
            pltpu.VMEM((2, CHUNK), jnp.int32),
            pltpu.VMEM((CHUNK,), jnp.int32),
            pltpu.VMEM((CHUNK,), jnp.int32),
            pltpu.VMEM((CHUNK, DH2), f32),
            pltpu.VMEM((CHUNK, DH2), f32),
            pltpu.VMEM((CHUNK, DH2 + 16), f32),
            pltpu.VMEM((CHUNK, DH2 + 16), f32),
            pltpu.VMEM((NP_SC * 2,), f32),
            pltpu.VMEM((NP_SC * 2,), f32),
            pltpu.VMEM_SHARED((NP_SC, DH2 + 16), f32),
            pltpu.SemaphoreType.DMA,
            pltpu.SemaphoreType.DMA,
            pltpu.SemaphoreType.DMA,
            pltpu.SemaphoreType.DMA,
            pltpu.SemaphoreType.DMA,
            pltpu.SemaphoreType.DMA,
        ],
    )
    za = jnp.zeros((ZROWS, DH2 + 16), f32)
    idx2 = jnp.stack([srcp, dstp], axis=2)
    aggz = sc_kernel(tsrc, tdst, wxsplit, idx2, za)
    aggz = jnp.concatenate(
        [aggz, jnp.zeros((2, NP - NP_SC, DH2 + 16), f32)], axis=1)
    agg = aggz[:, :, :DH2]
    z = aggz[:, :, DH2:]

    grid3 = (NP // NBLK,)
    out = pl.pallas_call(
        _tc2_body,
        grid=grid3,
        in_specs=[
            pl.BlockSpec((NBLK, DH2), lambda b: (b, 0)),
            pl.BlockSpec((NBLK, DH2), lambda b: (b, 0)),
            pl.BlockSpec((NBLK, 16), lambda b: (b, 0)),
            pl.BlockSpec((NBLK, 16), lambda b: (b, 0)),
            pl.BlockSpec((16, D), lambda b: (0, 0)),
            pl.BlockSpec((D, D), lambda b: (0, 0)),
            pl.BlockSpec((1, D), lambda b: (0, 0)),
        ],
        out_specs=pl.BlockSpec((NBLK, D), lambda b: (b, 0)),
        out_shape=jax.ShapeDtypeStruct((NP, D), f32),
    )(agg[0], agg[1], z[0], z[1], m16, Wo.T, bo.reshape(1, D))

    return out[:N_NODES]


# speedup vs baseline: 2.2768x; 1.8405x over previous
import dataclasses
import functools
import jax
import jax.numpy as jnp
import numpy as np
from jax import lax
from jax.experimental import pallas as pl
from jax.experimental.pallas import tpu as pltpu
from jax.experimental.pallas import tpu_sc as plsc

N_NODES = 10000
N_EDGES = 320000
D = 128
DH2 = 64
H = 4
D_H = 32

NP = 10240
NS = 16
CHUNK = 128
CHUNKS_PER_S = 158
EP = NS * CHUNKS_PER_S * CHUNK
NP_SC = 10016
ROWS_PER_TILE = NP_SC // NS
ZROWS = 313
NBLK = 256
LEAK = 0.2


def _tc1_body(x_ref, w_ref, asrc_ref, adst_ref, wx_ref, ssrc_ref, sdst_ref):
    b = pl.program_id(0)
    wx = jnp.dot(x_ref[...], w_ref[...], preferred_element_type=jnp.float32)
    ssrc = jnp.dot(wx, asrc_ref[...], preferred_element_type=jnp.float32)
    sdst = jnp.dot(wx, adst_ref[...], preferred_element_type=jnp.float32)
    rows = b * NBLK + lax.broadcasted_iota(jnp.int32, (NBLK, 1), 0)
    ssrc = jnp.where(rows < N_NODES, ssrc, jnp.float32(-1e30))
    wx_ref[...] = wx
    ssrc_ref[...] = ssrc
    sdst_ref[...] = sdst


def _tc2_body(a0_ref, a1_ref, z0_ref, z1_ref, m_ref, wot_ref, bo_ref, out_ref):
    agg = jnp.concatenate([a0_ref[...], a1_ref[...]], axis=1)
    z = z0_ref[...] + z1_ref[...]
    d = jnp.dot(z, m_ref[...], preferred_element_type=jnp.float32) + 1e-16
    o = jnp.dot(agg / d, wot_ref[...], preferred_element_type=jnp.float32)
    o = o + bo_ref[...]
    out_ref[...] = jnp.where(o > 0, o, jnp.exp(o) - 1.0)


def _sc_body(tsrc_h, tdst_h, wxs_h, idx_h, za_h,
             aggz_out,
             sd0, sd1, dscat0, dscat1, gwx0, gwx1, gz0, gz1,
             tsrc_v, tdst_v, aggz_s,
             isem0, isem1, gsem0, gsem1, ssem0, ssem1):
    c = lax.axis_index("c")
    s = lax.axis_index("s")
    sd = (sd0, sd1)
    dscat = (dscat0, dscat1)
    gwx = (gwx0, gwx1)
    gz = (gz0, gz1)
    isem = (isem0, isem1)
    gsem = (gsem0, gsem1)
    ssem = (ssem0, ssem1)

    @pl.loop(0, ROWS_PER_TILE // ZROWS)
    def _zero(r):
        base = s * ROWS_PER_TILE + r * ZROWS
        pltpu.sync_copy(za_h, aggz_s.at[pl.ds(base, ZROWS)])

    pltpu.sync_copy(tsrc_h.at[c], tsrc_v)
    pltpu.sync_copy(tdst_h.at[c], tdst_v)

    zeros16 = jnp.zeros((16,), jnp.float32)

    @pl.loop(0, CHUNK)
    def _wz(i):
        gz0[i, pl.ds(DH2, 16)] = zeros16
        gz1[i, pl.ds(DH2, 16)] = zeros16

    plsc.subcore_barrier()

    lane = lax.iota(jnp.int32, 16)
    h0 = c * 2
    lane_h = [jnp.full((16,), DH2 + h0 + hh, jnp.int32) for hh in range(2)]

    def _splat(v, idx):
        return lax.gather(
            v, idx[:, None],
            lax.GatherDimensionNumbers(
                offset_dims=(), collapsed_slice_dims=(0,),
                start_index_map=(0,)),
            (1,), mode=lax.GatherScatterMode.PROMISE_IN_BOUNDS)

    def _idx_copy(q, b):
        return pltpu.make_async_copy(idx_h.at[s].at[q], sd[b], isem[b])

    def _gather(q, b):
        return pltpu.make_async_copy(
            wxs_h.at[c].at[sd[b].at[0]], gwx[b], gsem[b])

    def _scatter(b):
        return pltpu.make_async_copy(gz[b], aggz_s.at[dscat[b]], ssem[b])

    cp0 = _idx_copy(0, 0)
    cp0.start()
    _idx_copy(1, 1).start()
    cp0.wait()
    _gather(0, 0).start()

    @pl.loop(0, CHUNKS_PER_S, step=2)
    def _chunk(cc):
        for b in range(2):
            q = cc + b
            nb = 1 - b

            @pl.when(q + 1 < CHUNKS_PER_S)
            def _prefetch():
                _idx_copy(q + 1, nb).wait()
                _gather(q + 1, nb).start()

            _gather(q, b).wait()

            @pl.when(q >= 2)
            def _drain():
                _scatter(b).wait()

            gwb = gwx[b]
            gzb = gz[b]
            for j in range(CHUNK // 16):
                s16 = sd[b][0, pl.ds(j * 16, 16)]
                d16 = sd[b][1, pl.ds(j * 16, 16)]
                dscat[b][pl.ds(j * 16, 16)] = d16
                s2 = s16 + s16
                d2 = d16 + d16
                wregs = []
                for hh in range(2):
                    siv = plsc.load_gather(tsrc_v, [s2 + hh])
                    sjv = plsc.load_gather(tdst_v, [d2 + hh])
                    es = siv + sjv
                    e = jnp.maximum(es, es * LEAK)
                    w16 = jnp.exp(e)
                    wregs.append(w16)
                    plsc.store_scatter(
                        gzb, [lane + j * 16, lane_h[hh]], w16)
                w0, w1 = wregs
                bcs = []
                for k in range(16):
                    ck = jnp.full((16,), k, jnp.int32)
                    bcs.append((_splat(w0, ck), _splat(w1, ck)))
                for k in range(16):
                    bc0, bc1 = bcs[k]
                    i = j * 16 + k
                    for hh, bc in ((0, bc0), (1, bc1)):
                        for qq in range(2):
                            sl = pl.ds(hh * 32 + qq * 16, 16)
                            gzb[i, sl] = gwb[i, sl] * bc

            pltpu.async_copy(gz[b], aggz_s.at[dscat[b]], ssem[b], add=True)

            @pl.when(q + 2 < CHUNKS_PER_S)
            def _nexti():
                _idx_copy(q + 2, b).start()

    _scatter(0).wait()
    _scatter(1).wait()

    plsc.subcore_barrier()
    stripe = pl.ds(s * ROWS_PER_TILE, ROWS_PER_TILE)
    pltpu.sync_copy(aggz_s.at[stripe], aggz_out.at[c, stripe])


def kernel(x, edge_index, W, attn_vec, Wo, bo):
    f32 = jnp.float32

    asrc = jnp.zeros((D, 16), f32)
    adst = jnp.zeros((D, 16), f32)
    for h in range(H):
        asrc = asrc.at[h * D_H:(h + 1) * D_H, h].set(attn_vec[h, :D_H])
        adst = adst.at[h * D_H:(h + 1) * D_H, h].set(attn_vec[h, D_H:])
    m16 = np.zeros((16, D), np.float32)
    for l in range(D):
        m16[l // D_H, l] = 1.0
    m16 = jnp.asarray(m16)

    xp = jnp.concatenate([x, jnp.zeros((NP - N_NODES, D), f32)])
    srcp = jnp.concatenate(
        [edge_index[0], jnp.full((EP - N_EDGES,), N_NODES, jnp.int32)]
    ).reshape(NS, CHUNKS_PER_S, CHUNK)
    dstp = jnp.concatenate(
        [edge_index[1], jnp.full((EP - N_EDGES,), N_NODES, jnp.int32)]
    ).reshape(NS, CHUNKS_PER_S, CHUNK)

    grid1 = (NP // NBLK,)
    wxp, ssrcp, sdstp = pl.pallas_call(
        _tc1_body,
        grid=grid1,
        in_specs=[
            pl.BlockSpec((NBLK, D), lambda b: (b, 0)),
            pl.BlockSpec((D, D), lambda b: (0, 0)),
            pl.BlockSpec((D, 16), lambda b: (0, 0)),
            pl.BlockSpec((D, 16), lambda b: (0, 0)),
        ],
        out_specs=[
            pl.BlockSpec((NBLK, D), lambda b: (b, 0)),
            pl.BlockSpec((NBLK, 16), lambda b: (b, 0)),
            pl.BlockSpec((NBLK, 16), lambda b: (b, 0)),
        ],
        out_shape=[
            jax.ShapeDtypeStruct((NP, D), f32),
            jax.ShapeDtypeStruct((NP, 16), f32),
            jax.ShapeDtypeStruct((NP, 16), f32),
        ],
    )(xp, W, asrc, adst)

    wxsplit = jnp.stack([wxp[:, :DH2], wxp[:, DH2:]])
    tsrc = jnp.stack([ssrcp[:NP_SC, 0:2].reshape(-1),
                      ssrcp[:NP_SC, 2:4].reshape(-1)])
    tdst = jnp.stack([sdstp[:NP_SC, 0:2].reshape(-1),
                      sdstp[:NP_SC, 2:4].reshape(-1)])

    mesh = plsc.VectorSubcoreMesh(core_axis_name="c", subcore_axis_name="s")
    cp = pltpu.CompilerParams(
        needs_layout_passes=False, use_tc_tiling_on_sc=False
    )
    sc_kernel = pl.kernel(
        _sc_body,
        compiler_params=cp,
        out_type=jax.ShapeDtypeStruct((2, NP_SC, DH2 + 16), f32),
        mesh=mesh,
        scratch_types=[
            pltpu.VMEM((2, CHUNK), jnp.int32),
            pltpu.VMEM((2, CHUNK), jnp.int32),
            pltpu.VMEM((CHUNK,), jnp.int32),
            pltpu.VMEM((CHUNK,), jnp.int32),
            pltpu.VMEM((CHUNK, DH2), f32),
            pltpu.VMEM((CHUNK, DH2), f32),
            pltpu.VMEM((CHUNK, DH2 + 16), f32),
            pltpu.VMEM((CHUNK, DH2 + 16), f32),
            pltpu.VMEM((NP_SC * 2,), f32),
            pltpu.VMEM((NP_SC * 2,), f32),
            pltpu.VMEM_SHARED((NP_SC, DH2 + 16), f32),
            pltpu.SemaphoreType.DMA,
            pltpu.SemaphoreType.DMA,
            pltpu.SemaphoreType.DMA,
            pltpu.SemaphoreType.DMA,
            pltpu.SemaphoreType.DMA,
            pltpu.SemaphoreType.DMA,
        ],
    )
    za = jnp.zeros((ZROWS, DH2 + 16), f32)
    idx2 = jnp.stack([srcp, dstp], axis=2)
    aggz = sc_kernel(tsrc, tdst, wxsplit, idx2, za)
    aggz = jnp.concatenate(
        [aggz, jnp.zeros((2, NP - NP_SC, DH2 + 16), f32)], axis=1)
    agg = aggz[:, :, :DH2]
    z = aggz[:, :, DH2:]

    grid3 = (NP // NBLK,)
    out = pl.pallas_call(
        _tc2_body,
        grid=grid3,
        in_specs=[
            pl.BlockSpec((NBLK, DH2), lambda b: (b, 0)),
            pl.BlockSpec((NBLK, DH2), lambda b: (b, 0)),
            pl.BlockSpec((NBLK, 16), lambda b: (b, 0)),
            pl.BlockSpec((NBLK, 16), lambda b: (b, 0)),
            pl.BlockSpec((16, D), lambda b: (0, 0)),
            pl.BlockSpec((D, D), lambda b: (0, 0)),
            pl.BlockSpec((1, D), lambda b: (0, 0)),
        ],
        out_specs=pl.BlockSpec((NBLK, D), lambda b: (b, 0)),
        out_shape=jax.ShapeDtypeStruct((NP, D), f32),
    )(agg[0], agg[1], z[0], z[1], m16, Wo.T, bo.reshape(1, D))

    return out[:N_NODES]


# speedup vs baseline: 2.4310x; 1.0677x over previous
import jax
import jax.numpy as jnp
import numpy as np
from jax import lax
from jax.experimental import pallas as pl
from jax.experimental.pallas import tpu as pltpu
from jax.experimental.pallas import tpu_sc as plsc

N_NODES = 10000
N_EDGES = 320000
D = 128
DH2 = 64
H = 4
D_H = 32

NP = 10240
NS = 16
CHUNK = 128
CHUNKS_PER_S = 158
EP = NS * CHUNKS_PER_S * CHUNK
NP_SC = 10016
ROWS_PER_TILE = NP_SC // NS
ZROWS = 313
NBLK = 256
NBLK3 = 2504
LEAK = 0.2


def _tc1_body(x_ref, w_ref, asrc_ref, adst_ref, wxs_ref, ssrc_ref, sdst_ref):
    b = pl.program_id(0)
    wx = jnp.dot(x_ref[...], w_ref[...], preferred_element_type=jnp.float32)
    ssrc = jnp.dot(wx, asrc_ref[...], preferred_element_type=jnp.float32)
    sdst = jnp.dot(wx, adst_ref[...], preferred_element_type=jnp.float32)
    rows = b * NBLK + lax.broadcasted_iota(jnp.int32, (NBLK, 1), 0)
    ssrc = jnp.where(rows < N_NODES, ssrc, jnp.float32(-1e30))
    wxs_ref[0] = wx[:, :DH2]
    wxs_ref[1] = wx[:, DH2:]
    ssrc_ref[...] = ssrc
    sdst_ref[...] = sdst


def _tc2_body(a0_ref, a1_ref, m_ref, wot_ref, bo_ref, out_ref):
    a0 = a0_ref[0]
    a1 = a1_ref[0]
    agg = jnp.concatenate([a0[:, :DH2], a1[:, :DH2]], axis=1)
    z = a0[:, DH2:] + a1[:, DH2:]
    d = jnp.dot(z, m_ref[...], preferred_element_type=jnp.float32) + 1e-16
    o = jnp.dot(agg / d, wot_ref[...], preferred_element_type=jnp.float32)
    o = o + bo_ref[...]
    out_ref[...] = jnp.where(o > 0, o, jnp.exp(o) - 1.0)


def _sc_body(tsrc_h, tdst_h, wxs_h, idx_h, za_h,
             aggz_out,
             sd0, sd1, dscat0, dscat1, gwx0, gwx1, gz0, gz1,
             tsrc_v, tdst_v, aggz_s,
             isem0, isem1, gsem0, gsem1, ssem0, ssem1):
    c = lax.axis_index("c")
    s = lax.axis_index("s")
    sd = (sd0, sd1)
    dscat = (dscat0, dscat1)
    gwx = (gwx0, gwx1)
    gz = (gz0, gz1)
    isem = (isem0, isem1)
    gsem = (gsem0, gsem1)
    ssem = (ssem0, ssem1)

    @pl.loop(0, ROWS_PER_TILE // ZROWS)
    def _zero(r):
        base = s * ROWS_PER_TILE + r * ZROWS
        pltpu.sync_copy(za_h, aggz_s.at[pl.ds(base, ZROWS)])

    pltpu.sync_copy(tsrc_h.at[c], tsrc_v)
    pltpu.sync_copy(tdst_h.at[c], tdst_v)

    zeros16 = jnp.zeros((16,), jnp.float32)

    @pl.loop(0, CHUNK)
    def _wz(i):
        gz0[i, pl.ds(DH2, 16)] = zeros16
        gz1[i, pl.ds(DH2, 16)] = zeros16

    plsc.subcore_barrier()

    lane = lax.iota(jnp.int32, 16)
    h0 = c * 2
    lane_h = [jnp.full((16,), DH2 + h0 + hh, jnp.int32) for hh in range(2)]

    def _splat(v, idx):
        return lax.gather(
            v, idx[:, None],
            lax.GatherDimensionNumbers(
                offset_dims=(), collapsed_slice_dims=(0,),
                start_index_map=(0,)),
            (1,), mode=lax.GatherScatterMode.PROMISE_IN_BOUNDS)

    def _idx_copy(q, b):
        return pltpu.make_async_copy(idx_h.at[s].at[q], sd[b], isem[b])

    def _gather(q, b):
        return pltpu.make_async_copy(
            wxs_h.at[c].at[sd[b].at[0]], gwx[b], gsem[b])

    def _scatter(b):
        return pltpu.make_async_copy(gz[b], aggz_s.at[dscat[b]], ssem[b])

    cp0 = _idx_copy(0, 0)
    cp0.start()
    _idx_copy(1, 1).start()
    cp0.wait()
    _gather(0, 0).start()

    @pl.loop(0, CHUNKS_PER_S, step=2)
    def _chunk(cc):
        for b in range(2):
            q = cc + b
            nb = 1 - b

            @pl.when(q + 1 < CHUNKS_PER_S)
            def _prefetch():
                _idx_copy(q + 1, nb).wait()
                _gather(q + 1, nb).start()

            _gather(q, b).wait()

            @pl.when(q >= 2)
            def _drain():
                _scatter(b).wait()

            gwb = gwx[b]
            gzb = gz[b]
            for j in range(CHUNK // 16):
                s16 = sd[b][0, pl.ds(j * 16, 16)]
                d16 = sd[b][1, pl.ds(j * 16, 16)]
                dscat[b][pl.ds(j * 16, 16)] = d16
                s2 = s16 + s16
                d2 = d16 + d16
                wregs = []
                for hh in range(2):
                    siv = plsc.load_gather(tsrc_v, [s2 + hh])
                    sjv = plsc.load_gather(tdst_v, [d2 + hh])
                    es = siv + sjv
                    e = jnp.maximum(es, es * LEAK)
                    w16 = jnp.exp(e)
                    wregs.append(w16)
                    plsc.store_scatter(
                        gzb, [lane + j * 16, lane_h[hh]], w16)
                w0, w1 = wregs
                for k0 in range(0, 16, 4):
                    bcs = []
                    for k in range(k0, k0 + 4):
                        ck = jnp.full((16,), k, jnp.int32)
                        bcs.append((_splat(w0, ck), _splat(w1, ck)))
                    for k in range(k0, k0 + 4):
                        bc0, bc1 = bcs[k - k0]
                        i = j * 16 + k
                        for hh, bc in ((0, bc0), (1, bc1)):
                            for qq in range(2):
                                sl = pl.ds(hh * 32 + qq * 16, 16)
                                gzb[i, sl] = gwb[i, sl] * bc

            pltpu.async_copy(gz[b], aggz_s.at[dscat[b]], ssem[b], add=True)

            @pl.when(q + 2 < CHUNKS_PER_S)
            def _nexti():
                _idx_copy(q + 2, b).start()

    _scatter(0).wait()
    _scatter(1).wait()

    plsc.subcore_barrier()
    stripe = pl.ds(s * ROWS_PER_TILE, ROWS_PER_TILE)
    pltpu.sync_copy(aggz_s.at[stripe], aggz_out.at[c, stripe])


def kernel(x, edge_index, W, attn_vec, Wo, bo):
    f32 = jnp.float32

    asrc = jnp.zeros((D, 16), f32)
    adst = jnp.zeros((D, 16), f32)
    for h in range(H):
        asrc = asrc.at[h * D_H:(h + 1) * D_H, h].set(attn_vec[h, :D_H])
        adst = adst.at[h * D_H:(h + 1) * D_H, h].set(attn_vec[h, D_H:])
    m16 = np.zeros((16, D), np.float32)
    for l in range(D):
        m16[l // D_H, l] = 1.0
    m16 = jnp.asarray(m16)

    xp = jnp.concatenate([x, jnp.zeros((NP - N_NODES, D), f32)])
    srcp = jnp.concatenate(
        [edge_index[0], jnp.full((EP - N_EDGES,), N_NODES, jnp.int32)]
    ).reshape(NS, CHUNKS_PER_S, CHUNK)
    dstp = jnp.concatenate(
        [edge_index[1], jnp.full((EP - N_EDGES,), N_NODES, jnp.int32)]
    ).reshape(NS, CHUNKS_PER_S, CHUNK)

    grid1 = (NP // NBLK,)
    wxsplit, ssrcp, sdstp = pl.pallas_call(
        _tc1_body,
        grid=grid1,
        in_specs=[
            pl.BlockSpec((NBLK, D), lambda b: (b, 0)),
            pl.BlockSpec((D, D), lambda b: (0, 0)),
            pl.BlockSpec((D, 16), lambda b: (0, 0)),
            pl.BlockSpec((D, 16), lambda b: (0, 0)),
        ],
        out_specs=[
            pl.BlockSpec((2, NBLK, DH2), lambda b: (0, b, 0)),
            pl.BlockSpec((NBLK, 16), lambda b: (b, 0)),
            pl.BlockSpec((NBLK, 16), lambda b: (b, 0)),
        ],
        out_shape=[
            jax.ShapeDtypeStruct((2, NP, DH2), f32),
            jax.ShapeDtypeStruct((NP, 16), f32),
            jax.ShapeDtypeStruct((NP, 16), f32),
        ],
    )(xp, W, asrc, adst)
    tsrc = jnp.stack([ssrcp[:NP_SC, 0:2].reshape(-1),
                      ssrcp[:NP_SC, 2:4].reshape(-1)])
    tdst = jnp.stack([sdstp[:NP_SC, 0:2].reshape(-1),
                      sdstp[:NP_SC, 2:4].reshape(-1)])

    mesh = plsc.VectorSubcoreMesh(core_axis_name="c", subcore_axis_name="s")
    cp = pltpu.CompilerParams(
        needs_layout_passes=False, use_tc_tiling_on_sc=False
    )
    sc_kernel = pl.kernel(
        _sc_body,
        compiler_params=cp,
        out_type=jax.ShapeDtypeStruct((2, NP_SC, DH2 + 16), f32),
        mesh=mesh,
        scratch_types=[
            pltpu.VMEM((2, CHUNK), jnp.int32),
            pltpu.VMEM((2, CHUNK), jnp.int32),
            pltpu.VMEM((CHUNK,), jnp.int32),
            pltpu.VMEM((CHUNK,), jnp.int32),
            pltpu.VMEM((CHUNK, DH2), f32),
            pltpu.VMEM((CHUNK, DH2), f32),
            pltpu.VMEM((CHUNK, DH2 + 16), f32),
            pltpu.VMEM((CHUNK, DH2 + 16), f32),
            pltpu.VMEM((NP_SC * 2,), f32),
            pltpu.VMEM((NP_SC * 2,), f32),
            pltpu.VMEM_SHARED((NP_SC, DH2 + 16), f32),
            pltpu.SemaphoreType.DMA,
            pltpu.SemaphoreType.DMA,
            pltpu.SemaphoreType.DMA,
            pltpu.SemaphoreType.DMA,
            pltpu.SemaphoreType.DMA,
            pltpu.SemaphoreType.DMA,
        ],
    )
    za = jnp.zeros((ZROWS, DH2 + 16), f32)
    idx2 = jnp.stack([srcp, dstp], axis=2)
    aggz = sc_kernel(tsrc, tdst, wxsplit, idx2, za)

    grid3 = (NP_SC // NBLK3,)
    out = pl.pallas_call(
        _tc2_body,
        grid=grid3,
        in_specs=[
            pl.BlockSpec((1, NBLK3, DH2 + 16), lambda b: (0, b, 0)),
            pl.BlockSpec((1, NBLK3, DH2 + 16), lambda b: (1, b, 0)),
            pl.BlockSpec((16, D), lambda b: (0, 0)),
            pl.BlockSpec((D, D), lambda b: (0, 0)),
            pl.BlockSpec((1, D), lambda b: (0, 0)),
        ],
        out_specs=pl.BlockSpec((NBLK3, D), lambda b: (b, 0)),
        out_shape=jax.ShapeDtypeStruct((NP_SC, D), f32),
    )(aggz, aggz, m16, Wo.T, bo.reshape(1, D))

    return out[:N_NODES]


# speedup vs baseline: 2.5228x; 1.0378x over previous
import jax
import jax.numpy as jnp
import numpy as np
from jax import lax
from jax.experimental import pallas as pl
from jax.experimental.pallas import tpu as pltpu
from jax.experimental.pallas import tpu_sc as plsc

N_NODES = 10000
N_EDGES = 320000
D = 128
DH2 = 64
H = 4
D_H = 32

NP = 10240
NS = 16
CHUNK = 128
CHUNKS_PER_S = 158
EP = NS * CHUNKS_PER_S * CHUNK
NP_SC = 10016
ROWS_PER_TILE = NP_SC // NS
ZROWS = 313
NBLK = 1024
NBLK3 = 10016
LEAK = 0.2


def _tc1_body(x_ref, w_ref, asrc_ref, adst_ref, wxs_ref, ssrc_ref, sdst_ref):
    b = pl.program_id(0)
    wx = jnp.dot(x_ref[...], w_ref[...], preferred_element_type=jnp.float32)
    ssrc = jnp.dot(wx, asrc_ref[...], preferred_element_type=jnp.float32)
    sdst = jnp.dot(wx, adst_ref[...], preferred_element_type=jnp.float32)
    rows = b * NBLK + lax.broadcasted_iota(jnp.int32, (NBLK, 1), 0)
    ssrc = jnp.where(rows < N_NODES, ssrc, jnp.float32(-1e30))
    wxs_ref[0] = wx[:, :DH2]
    wxs_ref[1] = wx[:, DH2:]
    ssrc_ref[...] = ssrc
    sdst_ref[...] = sdst


def _tc2_body(a0_ref, a1_ref, m_ref, wot_ref, bo_ref, out_ref):
    a0 = a0_ref[0]
    a1 = a1_ref[0]
    agg = jnp.concatenate([a0[:, :DH2], a1[:, :DH2]], axis=1)
    z = a0[:, DH2:] + a1[:, DH2:]
    d = jnp.dot(z, m_ref[...], preferred_element_type=jnp.float32) + 1e-16
    o = jnp.dot(agg / d, wot_ref[...], preferred_element_type=jnp.float32)
    o = o + bo_ref[...]
    out_ref[...] = jnp.where(o > 0, o, jnp.exp(o) - 1.0)


def _sc_body(tsrc_h, tdst_h, wxs_h, idx_h, za_h,
             aggz_out,
             sd0, sd1, dscat0, dscat1, gwx0, gwx1, gz0, gz1,
             tsrc_v, tdst_v, aggz_s,
             isem0, isem1, gsem0, gsem1, ssem0, ssem1):
    c = lax.axis_index("c")
    s = lax.axis_index("s")
    sd = (sd0, sd1)
    dscat = (dscat0, dscat1)
    gwx = (gwx0, gwx1)
    gz = (gz0, gz1)
    isem = (isem0, isem1)
    gsem = (gsem0, gsem1)
    ssem = (ssem0, ssem1)

    @pl.loop(0, ROWS_PER_TILE // ZROWS)
    def _zero(r):
        base = s * ROWS_PER_TILE + r * ZROWS
        pltpu.sync_copy(za_h, aggz_s.at[pl.ds(base, ZROWS)])

    pltpu.sync_copy(tsrc_h.at[c], tsrc_v)
    pltpu.sync_copy(tdst_h.at[c], tdst_v)

    zeros16 = jnp.zeros((16,), jnp.float32)

    @pl.loop(0, CHUNK)
    def _wz(i):
        gz0[i, pl.ds(DH2, 16)] = zeros16
        gz1[i, pl.ds(DH2, 16)] = zeros16

    plsc.subcore_barrier()

    lane = lax.iota(jnp.int32, 16)
    h0 = c * 2
    lane_h = [jnp.full((16,), DH2 + h0 + hh, jnp.int32) for hh in range(2)]

    def _splat(v, idx):
        return lax.gather(
            v, idx[:, None],
            lax.GatherDimensionNumbers(
                offset_dims=(), collapsed_slice_dims=(0,),
                start_index_map=(0,)),
            (1,), mode=lax.GatherScatterMode.PROMISE_IN_BOUNDS)

    def _idx_copy(q, b):
        return pltpu.make_async_copy(idx_h.at[s].at[q], sd[b], isem[b])

    def _gather(q, b):
        return pltpu.make_async_copy(
            wxs_h.at[c].at[sd[b].at[0]], gwx[b], gsem[b])

    def _scatter(b):
        return pltpu.make_async_copy(gz[b], aggz_s.at[dscat[b]], ssem[b])

    cp0 = _idx_copy(0, 0)
    cp0.start()
    _idx_copy(1, 1).start()
    cp0.wait()
    _gather(0, 0).start()

    @pl.loop(0, CHUNKS_PER_S, step=2)
    def _chunk(cc):
        for b in range(2):
            q = cc + b
            nb = 1 - b

            @pl.when(q + 1 < CHUNKS_PER_S)
            def _prefetch():
                _idx_copy(q + 1, nb).wait()
                _gather(q + 1, nb).start()

            _gather(q, b).wait()

            @pl.when(q >= 2)
            def _drain():
                _scatter(b).wait()

            gwb = gwx[b]
            gzb = gz[b]
            for j in range(CHUNK // 16):
                s16 = sd[b][0, pl.ds(j * 16, 16)]
                d16 = sd[b][1, pl.ds(j * 16, 16)]
                dscat[b][pl.ds(j * 16, 16)] = d16
                s2 = s16 + s16
                d2 = d16 + d16
                wregs = []
                for hh in range(2):
                    siv = plsc.load_gather(tsrc_v, [s2 + hh])
                    sjv = plsc.load_gather(tdst_v, [d2 + hh])
                    es = siv + sjv
                    e = jnp.maximum(es, es * LEAK)
                    w16 = jnp.exp(e)
                    wregs.append(w16)
                    plsc.store_scatter(
                        gzb, [lane + j * 16, lane_h[hh]], w16)
                w0, w1 = wregs
                for k0 in range(0, 16, 4):
                    bcs = []
                    for k in range(k0, k0 + 4):
                        ck = jnp.full((16,), k, jnp.int32)
                        bcs.append((_splat(w0, ck), _splat(w1, ck)))
                    for k in range(k0, k0 + 4):
                        bc0, bc1 = bcs[k - k0]
                        i = j * 16 + k
                        for hh, bc in ((0, bc0), (1, bc1)):
                            for qq in range(2):
                                sl = pl.ds(hh * 32 + qq * 16, 16)
                                gzb[i, sl] = gwb[i, sl] * bc

            pltpu.async_copy(gz[b], aggz_s.at[dscat[b]], ssem[b], add=True)

            @pl.when(q + 2 < CHUNKS_PER_S)
            def _nexti():
                _idx_copy(q + 2, b).start()

    _scatter(0).wait()
    _scatter(1).wait()

    plsc.subcore_barrier()
    stripe = pl.ds(s * ROWS_PER_TILE, ROWS_PER_TILE)
    pltpu.sync_copy(aggz_s.at[stripe], aggz_out.at[c, stripe])


def kernel(x, edge_index, W, attn_vec, Wo, bo):
    f32 = jnp.float32

    asrc = jnp.zeros((D, 16), f32)
    adst = jnp.zeros((D, 16), f32)
    for h in range(H):
        asrc = asrc.at[h * D_H:(h + 1) * D_H, h].set(attn_vec[h, :D_H])
        adst = adst.at[h * D_H:(h + 1) * D_H, h].set(attn_vec[h, D_H:])
    m16 = np.zeros((16, D), np.float32)
    for l in range(D):
        m16[l // D_H, l] = 1.0
    m16 = jnp.asarray(m16)

    xp = jnp.concatenate([x, jnp.zeros((NP - N_NODES, D), f32)])
    srcp = jnp.concatenate(
        [edge_index[0], jnp.full((EP - N_EDGES,), N_NODES, jnp.int32)]
    ).reshape(NS, CHUNKS_PER_S, CHUNK)
    dstp = jnp.concatenate(
        [edge_index[1], jnp.full((EP - N_EDGES,), N_NODES, jnp.int32)]
    ).reshape(NS, CHUNKS_PER_S, CHUNK)

    grid1 = (NP // NBLK,)
    wxsplit, ssrcp, sdstp = pl.pallas_call(
        _tc1_body,
        grid=grid1,
        in_specs=[
            pl.BlockSpec((NBLK, D), lambda b: (b, 0)),
            pl.BlockSpec((D, D), lambda b: (0, 0)),
            pl.BlockSpec((D, 16), lambda b: (0, 0)),
            pl.BlockSpec((D, 16), lambda b: (0, 0)),
        ],
        out_specs=[
            pl.BlockSpec((2, NBLK, DH2), lambda b: (0, b, 0)),
            pl.BlockSpec((NBLK, 16), lambda b: (b, 0)),
            pl.BlockSpec((NBLK, 16), lambda b: (b, 0)),
        ],
        out_shape=[
            jax.ShapeDtypeStruct((2, NP, DH2), f32),
            jax.ShapeDtypeStruct((NP, 16), f32),
            jax.ShapeDtypeStruct((NP, 16), f32),
        ],
    )(xp, W, asrc, adst)
    tsrc = jnp.stack([ssrcp[:NP_SC, 0:2].reshape(-1),
                      ssrcp[:NP_SC, 2:4].reshape(-1)])
    tdst = jnp.stack([sdstp[:NP_SC, 0:2].reshape(-1),
                      sdstp[:NP_SC, 2:4].reshape(-1)])

    mesh = plsc.VectorSubcoreMesh(core_axis_name="c", subcore_axis_name="s")
    cp = pltpu.CompilerParams(
        needs_layout_passes=False, use_tc_tiling_on_sc=False
    )
    sc_kernel = pl.kernel(
        _sc_body,
        compiler_params=cp,
        out_type=jax.ShapeDtypeStruct((2, NP_SC, DH2 + 16), f32),
        mesh=mesh,
        scratch_types=[
            pltpu.VMEM((2, CHUNK), jnp.int32),
            pltpu.VMEM((2, CHUNK), jnp.int32),
            pltpu.VMEM((CHUNK,), jnp.int32),
            pltpu.VMEM((CHUNK,), jnp.int32),
            pltpu.VMEM((CHUNK, DH2), f32),
            pltpu.VMEM((CHUNK, DH2), f32),
            pltpu.VMEM((CHUNK, DH2 + 16), f32),
            pltpu.VMEM((CHUNK, DH2 + 16), f32),
            pltpu.VMEM((NP_SC * 2,), f32),
            pltpu.VMEM((NP_SC * 2,), f32),
            pltpu.VMEM_SHARED((NP_SC, DH2 + 16), f32),
            pltpu.SemaphoreType.DMA,
            pltpu.SemaphoreType.DMA,
            pltpu.SemaphoreType.DMA,
            pltpu.SemaphoreType.DMA,
            pltpu.SemaphoreType.DMA,
            pltpu.SemaphoreType.DMA,
        ],
    )
    za = jnp.zeros((ZROWS, DH2 + 16), f32)
    idx2 = jnp.stack([srcp, dstp], axis=2)
    aggz = sc_kernel(tsrc, tdst, wxsplit, idx2, za)

    grid3 = (NP_SC // NBLK3,)
    out = pl.pallas_call(
        _tc2_body,
        grid=grid3,
        in_specs=[
            pl.BlockSpec((1, NBLK3, DH2 + 16), lambda b: (0, b, 0)),
            pl.BlockSpec((1, NBLK3, DH2 + 16), lambda b: (1, b, 0)),
            pl.BlockSpec((16, D), lambda b: (0, 0)),
            pl.BlockSpec((D, D), lambda b: (0, 0)),
            pl.BlockSpec((1, D), lambda b: (0, 0)),
        ],
        out_specs=pl.BlockSpec((NBLK3, D), lambda b: (b, 0)),
        out_shape=jax.ShapeDtypeStruct((NP_SC, D), f32),
    )(aggz, aggz, m16, Wo.T, bo.reshape(1, D))

    return out[:N_NODES]


# speedup vs baseline: 2.5267x; 1.0016x over previous
import jax
import jax.numpy as jnp
import numpy as np
from jax import lax
from jax.experimental import pallas as pl
from jax.experimental.pallas import tpu as pltpu
from jax.experimental.pallas import tpu_sc as plsc

N_NODES = 10000
N_EDGES = 320000
D = 128
DH2 = 64
H = 4
D_H = 32

NP = 10240
NS = 16
CHUNK = 128
CHUNKS_PER_S = 158
EP = NS * CHUNKS_PER_S * CHUNK
NP_SC = 10016
ROWS_PER_TILE = NP_SC // NS
ZROWS = 313
NBLK = 10240
NBLK3 = 10016
LEAK = 0.2


def _tc1_body(x_ref, w_ref, asrc_ref, adst_ref, wxs_ref, ssrc_ref, sdst_ref):
    b = pl.program_id(0)
    wx = jnp.dot(x_ref[...], w_ref[...], preferred_element_type=jnp.float32)
    ssrc = jnp.dot(wx, asrc_ref[...], preferred_element_type=jnp.float32)
    sdst = jnp.dot(wx, adst_ref[...], preferred_element_type=jnp.float32)
    rows = b * NBLK + lax.broadcasted_iota(jnp.int32, (NBLK, 1), 0)
    ssrc = jnp.where(rows < N_NODES, ssrc, jnp.float32(-1e30))
    wxs_ref[0] = wx[:, :DH2]
    wxs_ref[1] = wx[:, DH2:]
    ssrc_ref[...] = ssrc
    sdst_ref[...] = sdst


def _tc2_body(a0_ref, a1_ref, m_ref, wot_ref, bo_ref, out_ref):
    a0 = a0_ref[0]
    a1 = a1_ref[0]
    agg = jnp.concatenate([a0[:, :DH2], a1[:, :DH2]], axis=1)
    z = a0[:, DH2:] + a1[:, DH2:]
    d = jnp.dot(z, m_ref[...], preferred_element_type=jnp.float32) + 1e-16
    o = jnp.dot(agg / d, wot_ref[...], preferred_element_type=jnp.float32)
    o = o + bo_ref[...]
    out_ref[...] = jnp.where(o > 0, o, jnp.exp(o) - 1.0)


def _sc_body(tsrc_h, tdst_h, wxs_h, idx_h, za_h,
             aggz_out,
             sd0, sd1, dscat0, dscat1, gwx0, gwx1, gz0, gz1,
             tsrc_v, tdst_v, aggz_s,
             isem0, isem1, gsem0, gsem1, ssem0, ssem1):
    c = lax.axis_index("c")
    s = lax.axis_index("s")
    sd = (sd0, sd1)
    dscat = (dscat0, dscat1)
    gwx = (gwx0, gwx1)
    gz = (gz0, gz1)
    isem = (isem0, isem1)
    gsem = (gsem0, gsem1)
    ssem = (ssem0, ssem1)

    @pl.loop(0, ROWS_PER_TILE // ZROWS)
    def _zero(r):
        base = s * ROWS_PER_TILE + r * ZROWS
        pltpu.sync_copy(za_h, aggz_s.at[pl.ds(base, ZROWS)])

    pltpu.sync_copy(tsrc_h.at[c], tsrc_v)
    pltpu.sync_copy(tdst_h.at[c], tdst_v)

    zeros16 = jnp.zeros((16,), jnp.float32)

    @pl.loop(0, CHUNK)
    def _wz(i):
        gz0[i, pl.ds(DH2, 16)] = zeros16
        gz1[i, pl.ds(DH2, 16)] = zeros16

    plsc.subcore_barrier()

    lane = lax.iota(jnp.int32, 16)
    h0 = c * 2
    lane_h = [jnp.full((16,), DH2 + h0 + hh, jnp.int32) for hh in range(2)]

    def _splat(v, idx):
        return lax.gather(
            v, idx[:, None],
            lax.GatherDimensionNumbers(
                offset_dims=(), collapsed_slice_dims=(0,),
                start_index_map=(0,)),
            (1,), mode=lax.GatherScatterMode.PROMISE_IN_BOUNDS)

    def _idx_copy(q, b):
        return pltpu.make_async_copy(idx_h.at[s].at[q], sd[b], isem[b])

    def _gather(q, b):
        return pltpu.make_async_copy(
            wxs_h.at[c].at[sd[b].at[0]], gwx[b], gsem[b])

    def _scatter(b):
        return pltpu.make_async_copy(gz[b], aggz_s.at[dscat[b]], ssem[b])

    cp0 = _idx_copy(0, 0)
    cp0.start()
    _idx_copy(1, 1).start()
    cp0.wait()
    _gather(0, 0).start()

    @pl.loop(0, CHUNKS_PER_S, step=2)
    def _chunk(cc):
        for b in range(2):
            q = cc + b
            nb = 1 - b

            @pl.when(q + 1 < CHUNKS_PER_S)
            def _prefetch():
                _idx_copy(q + 1, nb).wait()
                _gather(q + 1, nb).start()

            _gather(q, b).wait()

            @pl.when(q >= 2)
            def _drain():
                _scatter(b).wait()

            gwb = gwx[b]
            gzb = gz[b]
            for j in range(CHUNK // 16):
                s16 = sd[b][0, pl.ds(j * 16, 16)]
                d16 = sd[b][1, pl.ds(j * 16, 16)]
                dscat[b][pl.ds(j * 16, 16)] = d16
                s2 = s16 + s16
                d2 = d16 + d16
                wregs = []
                for hh in range(2):
                    siv = plsc.load_gather(tsrc_v, [s2 + hh])
                    sjv = plsc.load_gather(tdst_v, [d2 + hh])
                    es = siv + sjv
                    e = jnp.maximum(es, es * LEAK)
                    w16 = jnp.exp(e)
                    wregs.append(w16)
                    plsc.store_scatter(
                        gzb, [lane + j * 16, lane_h[hh]], w16)
                w0, w1 = wregs
                for k0 in range(0, 16, 4):
                    bcs = []
                    for k in range(k0, k0 + 4):
                        ck = jnp.full((16,), k, jnp.int32)
                        bcs.append((_splat(w0, ck), _splat(w1, ck)))
                    for k in range(k0, k0 + 4):
                        bc0, bc1 = bcs[k - k0]
                        i = j * 16 + k
                        for hh, bc in ((0, bc0), (1, bc1)):
                            for qq in range(2):
                                sl = pl.ds(hh * 32 + qq * 16, 16)
                                gzb[i, sl] = gwb[i, sl] * bc

            pltpu.async_copy(gz[b], aggz_s.at[dscat[b]], ssem[b], add=True)

            @pl.when(q + 2 < CHUNKS_PER_S)
            def _nexti():
                _idx_copy(q + 2, b).start()

    _scatter(0).wait()
    _scatter(1).wait()

    plsc.subcore_barrier()
    stripe = pl.ds(s * ROWS_PER_TILE, ROWS_PER_TILE)
    pltpu.sync_copy(aggz_s.at[stripe], aggz_out.at[c, stripe])


def kernel(x, edge_index, W, attn_vec, Wo, bo):
    f32 = jnp.float32

    asrc = jnp.zeros((D, 16), f32)
    adst = jnp.zeros((D, 16), f32)
    for h in range(H):
        asrc = asrc.at[h * D_H:(h + 1) * D_H, h].set(attn_vec[h, :D_H])
        adst = adst.at[h * D_H:(h + 1) * D_H, h].set(attn_vec[h, D_H:])
    m16 = np.zeros((16, D), np.float32)
    for l in range(D):
        m16[l // D_H, l] = 1.0
    m16 = jnp.asarray(m16)

    xp = jnp.concatenate([x, jnp.zeros((NP - N_NODES, D), f32)])
    srcp = jnp.concatenate(
        [edge_index[0], jnp.full((EP - N_EDGES,), N_NODES, jnp.int32)]
    ).reshape(NS, CHUNKS_PER_S, CHUNK)
    dstp = jnp.concatenate(
        [edge_index[1], jnp.full((EP - N_EDGES,), N_NODES, jnp.int32)]
    ).reshape(NS, CHUNKS_PER_S, CHUNK)

    grid1 = (NP // NBLK,)
    wxsplit, ssrcp, sdstp = pl.pallas_call(
        _tc1_body,
        grid=grid1,
        in_specs=[
            pl.BlockSpec((NBLK, D), lambda b: (b, 0)),
            pl.BlockSpec((D, D), lambda b: (0, 0)),
            pl.BlockSpec((D, 16), lambda b: (0, 0)),
            pl.BlockSpec((D, 16), lambda b: (0, 0)),
        ],
        out_specs=[
            pl.BlockSpec((2, NBLK, DH2), lambda b: (0, b, 0)),
            pl.BlockSpec((NBLK, 16), lambda b: (b, 0)),
            pl.BlockSpec((NBLK, 16), lambda b: (b, 0)),
        ],
        out_shape=[
            jax.ShapeDtypeStruct((2, NP, DH2), f32),
            jax.ShapeDtypeStruct((NP, 16), f32),
            jax.ShapeDtypeStruct((NP, 16), f32),
        ],
    )(xp, W, asrc, adst)
    tsrc = jnp.stack([ssrcp[:NP_SC, 0:2].reshape(-1),
                      ssrcp[:NP_SC, 2:4].reshape(-1)])
    tdst = jnp.stack([sdstp[:NP_SC, 0:2].reshape(-1),
                      sdstp[:NP_SC, 2:4].reshape(-1)])

    mesh = plsc.VectorSubcoreMesh(core_axis_name="c", subcore_axis_name="s")
    cp = pltpu.CompilerParams(
        needs_layout_passes=False, use_tc_tiling_on_sc=False
    )
    sc_kernel = pl.kernel(
        _sc_body,
        compiler_params=cp,
        out_type=jax.ShapeDtypeStruct((2, NP_SC, DH2 + 16), f32),
        mesh=mesh,
        scratch_types=[
            pltpu.VMEM((2, CHUNK), jnp.int32),
            pltpu.VMEM((2, CHUNK), jnp.int32),
            pltpu.VMEM((CHUNK,), jnp.int32),
            pltpu.VMEM((CHUNK,), jnp.int32),
            pltpu.VMEM((CHUNK, DH2), f32),
            pltpu.VMEM((CHUNK, DH2), f32),
            pltpu.VMEM((CHUNK, DH2 + 16), f32),
            pltpu.VMEM((CHUNK, DH2 + 16), f32),
            pltpu.VMEM((NP_SC * 2,), f32),
            pltpu.VMEM((NP_SC * 2,), f32),
            pltpu.VMEM_SHARED((NP_SC, DH2 + 16), f32),
            pltpu.SemaphoreType.DMA,
            pltpu.SemaphoreType.DMA,
            pltpu.SemaphoreType.DMA,
            pltpu.SemaphoreType.DMA,
            pltpu.SemaphoreType.DMA,
            pltpu.SemaphoreType.DMA,
        ],
    )
    za = jnp.zeros((ZROWS, DH2 + 16), f32)
    idx2 = jnp.stack([srcp, dstp], axis=2)
    aggz = sc_kernel(tsrc, tdst, wxsplit, idx2, za)

    grid3 = (NP_SC // NBLK3,)
    out = pl.pallas_call(
        _tc2_body,
        grid=grid3,
        in_specs=[
            pl.BlockSpec((1, NBLK3, DH2 + 16), lambda b: (0, b, 0)),
            pl.BlockSpec((1, NBLK3, DH2 + 16), lambda b: (1, b, 0)),
            pl.BlockSpec((16, D), lambda b: (0, 0)),
            pl.BlockSpec((D, D), lambda b: (0, 0)),
            pl.BlockSpec((1, D), lambda b: (0, 0)),
        ],
        out_specs=pl.BlockSpec((NBLK3, D), lambda b: (b, 0)),
        out_shape=jax.ShapeDtypeStruct((NP_SC, D), f32),
    )(aggz, aggz, m16, Wo.T, bo.reshape(1, D))

    return out[:N_NODES]
